# baseline (speedup 1.0000x reference)
import functools

import jax
import jax.numpy as jnp
from jax import lax
from jax.experimental import pallas as pl
from jax.experimental.pallas import tpu as pltpu
from jax.experimental.pallas import tpu_sc as plsc

D = 128
B = 1024
S = 200
L = 16
NC, NS = 2, 16
NW = NC * NS
ROWS_PER_W = B // NW
G0, G1 = 96, 104
W0, W1 = 16, 184
FLAT = B * S


@jax.jit
def _sc_embed(x_flat, token_table, pos_table):
    mesh = plsc.VectorSubcoreMesh(core_axis_name="c", subcore_axis_name="s")

    @functools.partial(
        pl.kernel,
        mesh=mesh,
        out_type=jax.ShapeDtypeStruct((FLAT, D), jnp.float32),
        scratch_types=[
            pltpu.VMEM((S * ROWS_PER_W,), jnp.int32),
            pltpu.VMEM((S, D), jnp.float32),
            pltpu.VMEM_SHARED((NS, 2, W1, D), jnp.float32),
            pltpu.VMEM((S, D), jnp.float32),
            pltpu.VMEM((S, D), jnp.float32),
            pltpu.SemaphoreType.DMA,
            pltpu.SemaphoreType.DMA,
            pltpu.SemaphoreType.DMA,
            pltpu.SemaphoreType.DMA,
            pltpu.SemaphoreType.DMA,
            pltpu.SemaphoreType.DMA,
            pltpu.SemaphoreType.DMA,
            pltpu.SemaphoreType.DMA,
            pltpu.SemaphoreType.DMA,
        ],
    )
    def k(tok_hbm, pos_hbm, idx_hbm, out_hbm, idx_v, pos_v, shared,
          buf0, buf1, gsem0, gsem1, s1sem0, s1sem1, odsem0, odsem1,
          s2sem0, s2sem1, psem):
        bufs = (buf0, buf1)
        gsem = (gsem0, gsem1)
        s1sem = (s1sem0, s1sem1)
        odsem = (odsem0, odsem1)
        s2sem = (s2sem0, s2sem1)

        sid = lax.axis_index("s")
        wid = sid * NC + lax.axis_index("c")
        wbase = wid * (S * ROWS_PER_W)
        pltpu.sync_copy(idx_hbm.at[pl.ds(wbase, S * ROWS_PER_W)], idx_v)
        pos_copy = pltpu.async_copy(pos_hbm, pos_v, psem)

        def fire_gather(r, kb):
            pltpu.async_copy(
                tok_hbm.at[idx_v.at[pl.ds(r * S, G0)]],
                bufs[kb].at[pl.ds(0, G0)], gsem[kb])
            pltpu.async_copy(
                tok_hbm.at[idx_v.at[pl.ds(r * S + G0, G1)]],
                bufs[kb].at[pl.ds(G0, G1)], gsem[kb])

        def drain_gather(kb):
            pltpu.make_async_copy(
                tok_hbm.at[pl.ds(0, S)], bufs[kb], gsem[kb]).wait()

        def fire_split_out(r, kb):
            pltpu.async_copy(
                bufs[kb].at[pl.ds(0, W0)],
                out_hbm.at[pl.ds(wbase + r * S, W0)], odsem[kb])
            pltpu.async_copy(
                bufs[kb].at[pl.ds(W0, W1)], shared.at[sid, kb], s1sem[kb])

        def drain_split_out(kb):
            pltpu.make_async_copy(
                bufs[kb].at[pl.ds(0, W0)],
                out_hbm.at[pl.ds(0, W0)], odsem[kb]).wait()
            pltpu.make_async_copy(
                bufs[kb].at[pl.ds(W0, W1)], shared.at[sid, kb],
                s1sem[kb]).wait()

        def fire_s2(r, kb):
            pltpu.async_copy(
                shared.at[sid, kb],
                out_hbm.at[pl.ds(wbase + r * S + W0, W1)], s2sem[kb])

        def drain_s2(kb):
            pltpu.make_async_copy(
                shared.at[sid, kb], out_hbm.at[pl.ds(0, W1)],
                s2sem[kb]).wait()

        def add_pos(kb):
            buf = bufs[kb]

            @pl.loop(0, S)
            def _(i):
                for c in range(0, D, L):
                    plsc.addupdate(buf.at[i, pl.ds(c, L)],
                                   pos_v[i, pl.ds(c, L)])

        fire_gather(0, 0)
        pos_copy.wait()

        @pl.loop(0, ROWS_PER_W + 2, step=2)
        def _(r0):
            for kb in range(2):
                r = r0 + kb
                kp = 1 - kb
                cond = (r >= 1) & (r < ROWS_PER_W + 1)

                @pl.when(cond)
                def _():
                    drain_split_out(kp)
                    fire_s2(r - 1, kp)

                @pl.when(r + 1 < ROWS_PER_W)
                def _():
                    fire_gather(r + 1, kp)

                @pl.when(r < ROWS_PER_W)
                def _():
                    drain_gather(kb)
                    add_pos(kb)

                    @pl.when(r >= 2)
                    def _():
                        drain_s2(kb)
                    fire_split_out(r, kb)

        drain_s2(0)
        drain_s2(1)

    return k(token_table, pos_table, x_flat)


def kernel(x, token_table, pos_table):
    x_flat = x.reshape(FLAT).astype(jnp.int32)
    out = _sc_embed(x_flat, token_table, pos_table)
    return out.reshape(B, S, D)


# speedup vs baseline: 1.0606x; 1.0606x over previous
import functools

import jax
import jax.numpy as jnp
from jax import lax
from jax.experimental import pallas as pl
from jax.experimental.pallas import tpu as pltpu
from jax.experimental.pallas import tpu_sc as plsc

D = 128
B = 1024
S = 200
L = 16
NC, NS = 2, 16
NW = NC * NS
ROWS_PER_W = B // NW
G0, G1 = 96, 104
W0, W1 = 64, 136
FLAT = B * S


@jax.jit
def _sc_embed(x_flat, token_table, pos_table):
    mesh = plsc.VectorSubcoreMesh(core_axis_name="c", subcore_axis_name="s")

    @functools.partial(
        pl.kernel,
        mesh=mesh,
        out_type=jax.ShapeDtypeStruct((FLAT, D), jnp.float32),
        scratch_types=[
            pltpu.VMEM((S * ROWS_PER_W,), jnp.int32),
            pltpu.VMEM((S, D), jnp.float32),
            pltpu.VMEM_SHARED((NS, 2, W1, D), jnp.float32),
            pltpu.VMEM((S, D), jnp.float32),
            pltpu.VMEM((S, D), jnp.float32),
            pltpu.SemaphoreType.DMA,
            pltpu.SemaphoreType.DMA,
            pltpu.SemaphoreType.DMA,
            pltpu.SemaphoreType.DMA,
            pltpu.SemaphoreType.DMA,
            pltpu.SemaphoreType.DMA,
            pltpu.SemaphoreType.DMA,
            pltpu.SemaphoreType.DMA,
            pltpu.SemaphoreType.DMA,
        ],
    )
    def k(tok_hbm, pos_hbm, idx_hbm, out_hbm, idx_v, pos_v, shared,
          buf0, buf1, gsem0, gsem1, s1sem0, s1sem1, odsem0, odsem1,
          s2sem0, s2sem1, psem):
        bufs = (buf0, buf1)
        gsem = (gsem0, gsem1)
        s1sem = (s1sem0, s1sem1)
        odsem = (odsem0, odsem1)
        s2sem = (s2sem0, s2sem1)

        sid = lax.axis_index("s")
        wid = sid * NC + lax.axis_index("c")
        wbase = wid * (S * ROWS_PER_W)
        pltpu.sync_copy(idx_hbm.at[pl.ds(wbase, S * ROWS_PER_W)], idx_v)
        pos_copy = pltpu.async_copy(pos_hbm, pos_v, psem)

        def fire_gather(r, kb):
            pltpu.async_copy(
                tok_hbm.at[idx_v.at[pl.ds(r * S, G0)]],
                bufs[kb].at[pl.ds(0, G0)], gsem[kb])
            pltpu.async_copy(
                tok_hbm.at[idx_v.at[pl.ds(r * S + G0, G1)]],
                bufs[kb].at[pl.ds(G0, G1)], gsem[kb])

        def drain_gather(kb):
            pltpu.make_async_copy(
                tok_hbm.at[pl.ds(0, S)], bufs[kb], gsem[kb]).wait()

        def fire_split_out(r, kb):
            pltpu.async_copy(
                bufs[kb].at[pl.ds(0, W0)],
                out_hbm.at[pl.ds(wbase + r * S, W0)], odsem[kb])
            pltpu.async_copy(
                bufs[kb].at[pl.ds(W0, W1)], shared.at[sid, kb], s1sem[kb])

        def drain_split_out(kb):
            pltpu.make_async_copy(
                bufs[kb].at[pl.ds(0, W0)],
                out_hbm.at[pl.ds(0, W0)], odsem[kb]).wait()
            pltpu.make_async_copy(
                bufs[kb].at[pl.ds(W0, W1)], shared.at[sid, kb],
                s1sem[kb]).wait()

        def fire_s2(r, kb):
            pltpu.async_copy(
                shared.at[sid, kb],
                out_hbm.at[pl.ds(wbase + r * S + W0, W1)], s2sem[kb])

        def drain_s2(kb):
            pltpu.make_async_copy(
                shared.at[sid, kb], out_hbm.at[pl.ds(0, W1)],
                s2sem[kb]).wait()

        def add_pos(kb):
            buf = bufs[kb]

            @pl.loop(0, S)
            def _(i):
                for c in range(0, D, L):
                    plsc.addupdate(buf.at[i, pl.ds(c, L)],
                                   pos_v[i, pl.ds(c, L)])

        fire_gather(0, 0)
        pos_copy.wait()

        @pl.loop(0, ROWS_PER_W + 2, step=2)
        def _(r0):
            for kb in range(2):
                r = r0 + kb
                kp = 1 - kb
                cond = (r >= 1) & (r < ROWS_PER_W + 1)

                @pl.when(cond)
                def _():
                    drain_split_out(kp)
                    fire_s2(r - 1, kp)

                @pl.when(r + 1 < ROWS_PER_W)
                def _():
                    fire_gather(r + 1, kp)

                @pl.when(r < ROWS_PER_W)
                def _():
                    drain_gather(kb)
                    add_pos(kb)

                    @pl.when(r >= 2)
                    def _():
                        drain_s2(kb)
                    fire_split_out(r, kb)

        drain_s2(0)
        drain_s2(1)

    return k(token_table, pos_table, x_flat)


def kernel(x, token_table, pos_table):
    x_flat = x.reshape(FLAT).astype(jnp.int32)
    out = _sc_embed(x_flat, token_table, pos_table)
    return out.reshape(B, S, D)


# speedup vs baseline: 1.0865x; 1.0244x over previous
import functools

import jax
import jax.numpy as jnp
from jax import lax
from jax.experimental import pallas as pl
from jax.experimental.pallas import tpu as pltpu
from jax.experimental.pallas import tpu_sc as plsc

D = 128
B = 1024
S = 200
L = 16
NC, NS = 2, 16
NW = NC * NS
ROWS_PER_W = B // NW
G0, G1 = 96, 104
W0, W1 = 88, 112
FLAT = B * S


@jax.jit
def _sc_embed(x_flat, token_table, pos_table):
    mesh = plsc.VectorSubcoreMesh(core_axis_name="c", subcore_axis_name="s")

    @functools.partial(
        pl.kernel,
        mesh=mesh,
        out_type=jax.ShapeDtypeStruct((FLAT, D), jnp.float32),
        scratch_types=[
            pltpu.VMEM((S * ROWS_PER_W,), jnp.int32),
            pltpu.VMEM((S, D), jnp.float32),
            pltpu.VMEM_SHARED((NS, 2, W1, D), jnp.float32),
            pltpu.VMEM((S, D), jnp.float32),
            pltpu.VMEM((S, D), jnp.float32),
            pltpu.SemaphoreType.DMA,
            pltpu.SemaphoreType.DMA,
            pltpu.SemaphoreType.DMA,
            pltpu.SemaphoreType.DMA,
            pltpu.SemaphoreType.DMA,
            pltpu.SemaphoreType.DMA,
            pltpu.SemaphoreType.DMA,
            pltpu.SemaphoreType.DMA,
            pltpu.SemaphoreType.DMA,
        ],
    )
    def k(tok_hbm, pos_hbm, idx_hbm, out_hbm, idx_v, pos_v, shared,
          buf0, buf1, gsem0, gsem1, s1sem0, s1sem1, odsem0, odsem1,
          s2sem0, s2sem1, psem):
        bufs = (buf0, buf1)
        gsem = (gsem0, gsem1)
        s1sem = (s1sem0, s1sem1)
        odsem = (odsem0, odsem1)
        s2sem = (s2sem0, s2sem1)

        sid = lax.axis_index("s")
        wid = sid * NC + lax.axis_index("c")
        wbase = wid * (S * ROWS_PER_W)
        pltpu.sync_copy(idx_hbm.at[pl.ds(wbase, S * ROWS_PER_W)], idx_v)
        pos_copy = pltpu.async_copy(pos_hbm, pos_v, psem)

        def fire_gather(r, kb):
            pltpu.async_copy(
                tok_hbm.at[idx_v.at[pl.ds(r * S, G0)]],
                bufs[kb].at[pl.ds(0, G0)], gsem[kb])
            pltpu.async_copy(
                tok_hbm.at[idx_v.at[pl.ds(r * S + G0, G1)]],
                bufs[kb].at[pl.ds(G0, G1)], gsem[kb])

        def drain_gather(kb):
            pltpu.make_async_copy(
                tok_hbm.at[pl.ds(0, S)], bufs[kb], gsem[kb]).wait()

        def fire_split_out(r, kb):
            pltpu.async_copy(
                bufs[kb].at[pl.ds(0, W0)],
                out_hbm.at[pl.ds(wbase + r * S, W0)], odsem[kb])
            pltpu.async_copy(
                bufs[kb].at[pl.ds(W0, W1)], shared.at[sid, kb], s1sem[kb])

        def drain_split_out(kb):
            pltpu.make_async_copy(
                bufs[kb].at[pl.ds(0, W0)],
                out_hbm.at[pl.ds(0, W0)], odsem[kb]).wait()
            pltpu.make_async_copy(
                bufs[kb].at[pl.ds(W0, W1)], shared.at[sid, kb],
                s1sem[kb]).wait()

        def fire_s2(r, kb):
            pltpu.async_copy(
                shared.at[sid, kb],
                out_hbm.at[pl.ds(wbase + r * S + W0, W1)], s2sem[kb])

        def drain_s2(kb):
            pltpu.make_async_copy(
                shared.at[sid, kb], out_hbm.at[pl.ds(0, W1)],
                s2sem[kb]).wait()

        def add_pos(kb):
            buf = bufs[kb]

            @pl.loop(0, S)
            def _(i):
                for c in range(0, D, L):
                    plsc.addupdate(buf.at[i, pl.ds(c, L)],
                                   pos_v[i, pl.ds(c, L)])

        fire_gather(0, 0)
        pos_copy.wait()

        @pl.loop(0, ROWS_PER_W + 2, step=2)
        def _(r0):
            for kb in range(2):
                r = r0 + kb
                kp = 1 - kb
                cond = (r >= 1) & (r < ROWS_PER_W + 1)

                @pl.when(cond)
                def _():
                    drain_split_out(kp)
                    fire_s2(r - 1, kp)

                @pl.when(r + 1 < ROWS_PER_W)
                def _():
                    fire_gather(r + 1, kp)

                @pl.when(r < ROWS_PER_W)
                def _():
                    drain_gather(kb)
                    add_pos(kb)

                    @pl.when(r >= 2)
                    def _():
                        drain_s2(kb)
                    fire_split_out(r, kb)

        drain_s2(0)
        drain_s2(1)

    return k(token_table, pos_table, x_flat)


def kernel(x, token_table, pos_table):
    x_flat = x.reshape(FLAT).astype(jnp.int32)
    out = _sc_embed(x_flat, token_table, pos_table)
    return out.reshape(B, S, D)


# speedup vs baseline: 1.1050x; 1.0170x over previous
import functools

import jax
import jax.numpy as jnp
from jax import lax
from jax.experimental import pallas as pl
from jax.experimental.pallas import tpu as pltpu
from jax.experimental.pallas import tpu_sc as plsc

D = 128
B = 1024
S = 200
L = 16
NC, NS = 2, 16
NW = NC * NS
ROWS_PER_W = B // NW
G0, G1 = 96, 104
W0, W1 = 88, 112
FLAT = B * S


@jax.jit
def _sc_embed(x_flat, token_table, pos_table):
    mesh = plsc.VectorSubcoreMesh(core_axis_name="c", subcore_axis_name="s")

    @functools.partial(
        pl.kernel,
        mesh=mesh,
        out_type=jax.ShapeDtypeStruct((FLAT, D), jnp.float32),
        scratch_types=[
            pltpu.VMEM((S * ROWS_PER_W,), jnp.int32),
            pltpu.VMEM((S, D), jnp.float32),
            pltpu.VMEM_SHARED((NS, 2, W1, D), jnp.float32),
            pltpu.VMEM((S, D), jnp.float32),
            pltpu.VMEM((S, D), jnp.float32),
            pltpu.SemaphoreType.DMA,
            pltpu.SemaphoreType.DMA,
            pltpu.SemaphoreType.DMA,
            pltpu.SemaphoreType.DMA,
            pltpu.SemaphoreType.DMA,
            pltpu.SemaphoreType.DMA,
            pltpu.SemaphoreType.DMA,
            pltpu.SemaphoreType.DMA,
            pltpu.SemaphoreType.DMA,
        ],
    )
    def k(tok_hbm, pos_hbm, idx_hbm, out_hbm, idx_v, pos_v, shared,
          buf0, buf1, gsem0, gsem1, s1sem0, s1sem1, odsem0, odsem1,
          s2sem0, s2sem1, psem):
        bufs = (buf0, buf1)
        gsem = (gsem0, gsem1)
        s1sem = (s1sem0, s1sem1)
        odsem = (odsem0, odsem1)
        s2sem = (s2sem0, s2sem1)

        sid = lax.axis_index("s")
        wid = sid * NC + lax.axis_index("c")
        wbase = wid * (S * ROWS_PER_W)
        pltpu.sync_copy(idx_hbm.at[pl.ds(wbase, S * ROWS_PER_W)], idx_v)
        pos_copy = pltpu.async_copy(pos_hbm, pos_v, psem)

        def fire_gather(r, kb):
            pltpu.async_copy(
                tok_hbm.at[idx_v.at[pl.ds(r * S, G0)]],
                bufs[kb].at[pl.ds(0, G0)], gsem[kb])
            pltpu.async_copy(
                tok_hbm.at[idx_v.at[pl.ds(r * S + G0, G1)]],
                bufs[kb].at[pl.ds(G0, G1)], gsem[kb])

        def drain_gather(kb):
            pltpu.make_async_copy(
                tok_hbm.at[pl.ds(0, S)], bufs[kb], gsem[kb]).wait()

        def fire_split_out(r, kb):
            pltpu.async_copy(
                bufs[kb].at[pl.ds(0, W0)],
                out_hbm.at[pl.ds(wbase + r * S, W0)], odsem[kb])
            pltpu.async_copy(
                bufs[kb].at[pl.ds(W0, W1)], shared.at[sid, kb], s1sem[kb])

        def drain_split_out(kb):
            pltpu.make_async_copy(
                bufs[kb].at[pl.ds(0, W0)],
                out_hbm.at[pl.ds(0, W0)], odsem[kb]).wait()
            pltpu.make_async_copy(
                bufs[kb].at[pl.ds(W0, W1)], shared.at[sid, kb],
                s1sem[kb]).wait()

        def fire_s2(r, kb):
            pltpu.async_copy(
                shared.at[sid, kb],
                out_hbm.at[pl.ds(wbase + r * S + W0, W1)], s2sem[kb])

        def drain_s2(kb):
            pltpu.make_async_copy(
                shared.at[sid, kb], out_hbm.at[pl.ds(0, W1)],
                s2sem[kb]).wait()

        def add_pos(kb, lo, hi):
            buf = bufs[kb]

            @pl.loop(lo, hi)
            def _(i):
                for c in range(0, D, L):
                    plsc.addupdate(buf.at[i, pl.ds(c, L)],
                                   pos_v[i, pl.ds(c, L)])

        fire_gather(0, 0)
        pos_copy.wait()

        @pl.loop(0, ROWS_PER_W + 2, step=2)
        def _(r0):
            for kb in range(2):
                r = r0 + kb
                kp = 1 - kb
                cond = (r >= 1) & (r < ROWS_PER_W + 1)

                @pl.when(cond)
                def _():
                    drain_split_out(kp)
                    fire_s2(r - 1, kp)

                @pl.when(r + 1 < ROWS_PER_W)
                def _():
                    fire_gather(r + 1, kp)

                @pl.when(r < ROWS_PER_W)
                def _():
                    drain_gather(kb)
                    add_pos(kb, 0, W0)
                    pltpu.async_copy(
                        bufs[kb].at[pl.ds(0, W0)],
                        out_hbm.at[pl.ds(wbase + r * S, W0)], odsem[kb])
                    add_pos(kb, W0, S)

                    @pl.when(r >= 2)
                    def _():
                        drain_s2(kb)
                    pltpu.async_copy(
                        bufs[kb].at[pl.ds(W0, W1)], shared.at[sid, kb],
                        s1sem[kb])

        drain_s2(0)
        drain_s2(1)

    return k(token_table, pos_table, x_flat)


def kernel(x, token_table, pos_table):
    x_flat = x.reshape(FLAT).astype(jnp.int32)
    out = _sc_embed(x_flat, token_table, pos_table)
    return out.reshape(B, S, D)


# speedup vs baseline: 1.1052x; 1.0002x over previous
import functools

import jax
import jax.numpy as jnp
from jax import lax
from jax.experimental import pallas as pl
from jax.experimental.pallas import tpu as pltpu
from jax.experimental.pallas import tpu_sc as plsc

D = 128
B = 1024
S = 200
L = 16
NC, NS = 2, 16
NW = NC * NS
ROWS_PER_W = B // NW
G0, G1 = 96, 104
W0, W1 = 88, 112
FLAT = B * S


@jax.jit
def _sc_embed(x_flat, token_table, pos_table):
    mesh = plsc.VectorSubcoreMesh(core_axis_name="c", subcore_axis_name="s")

    @functools.partial(
        pl.kernel,
        mesh=mesh,
        out_type=jax.ShapeDtypeStruct((FLAT, D), jnp.float32),
        scratch_types=[
            pltpu.VMEM((S * ROWS_PER_W,), jnp.int32),
            pltpu.VMEM((S, D), jnp.float32),
            pltpu.VMEM_SHARED((NS, 2, W1, D), jnp.float32),
            pltpu.VMEM((S, D), jnp.float32),
            pltpu.VMEM((S, D), jnp.float32),
            pltpu.SemaphoreType.DMA,
            pltpu.SemaphoreType.DMA,
            pltpu.SemaphoreType.DMA,
            pltpu.SemaphoreType.DMA,
            pltpu.SemaphoreType.DMA,
            pltpu.SemaphoreType.DMA,
            pltpu.SemaphoreType.DMA,
            pltpu.SemaphoreType.DMA,
            pltpu.SemaphoreType.DMA,
            pltpu.SemaphoreType.DMA,
            pltpu.SemaphoreType.DMA,
        ],
    )
    def k(tok_hbm, pos_hbm, idx_hbm, out_hbm, idx_v, pos_v, shared,
          buf0, buf1, gsem0, gsem1, gbsem0, gbsem1, s1sem0, s1sem1,
          odsem0, odsem1, s2sem0, s2sem1, psem):
        bufs = (buf0, buf1)
        gsem = (gsem0, gsem1)
        gbsem = (gbsem0, gbsem1)
        s1sem = (s1sem0, s1sem1)
        odsem = (odsem0, odsem1)
        s2sem = (s2sem0, s2sem1)

        sid = lax.axis_index("s")
        wid = sid * NC + lax.axis_index("c")
        wbase = wid * (S * ROWS_PER_W)
        pltpu.sync_copy(idx_hbm.at[pl.ds(wbase, S * ROWS_PER_W)], idx_v)
        pos_copy = pltpu.async_copy(pos_hbm, pos_v, psem)

        def fire_gather(r, kb):
            pltpu.async_copy(
                tok_hbm.at[idx_v.at[pl.ds(r * S, G0)]],
                bufs[kb].at[pl.ds(0, G0)], gsem[kb])
            pltpu.async_copy(
                tok_hbm.at[idx_v.at[pl.ds(r * S + G0, G1)]],
                bufs[kb].at[pl.ds(G0, G1)], gbsem[kb])

        def drain_gather_a(kb):
            pltpu.make_async_copy(
                tok_hbm.at[pl.ds(0, G0)], bufs[kb].at[pl.ds(0, G0)],
                gsem[kb]).wait()

        def drain_gather_b(kb):
            pltpu.make_async_copy(
                tok_hbm.at[pl.ds(0, G1)], bufs[kb].at[pl.ds(G0, G1)],
                gbsem[kb]).wait()

        def drain_od(kb):
            pltpu.make_async_copy(
                bufs[kb].at[pl.ds(0, W0)],
                out_hbm.at[pl.ds(0, W0)], odsem[kb]).wait()

        def drain_s1(kb):
            pltpu.make_async_copy(
                bufs[kb].at[pl.ds(W0, W1)], shared.at[sid, kb],
                s1sem[kb]).wait()

        def fire_s2(r, kb):
            pltpu.async_copy(
                shared.at[sid, kb],
                out_hbm.at[pl.ds(wbase + r * S + W0, W1)], s2sem[kb])

        def drain_s2(kb):
            pltpu.make_async_copy(
                shared.at[sid, kb], out_hbm.at[pl.ds(0, W1)],
                s2sem[kb]).wait()

        def add_pos(kb, lo, hi):
            buf = bufs[kb]

            @pl.loop(lo, hi)
            def _(i):
                for c in range(0, D, L):
                    plsc.addupdate(buf.at[i, pl.ds(c, L)],
                                   pos_v[i, pl.ds(c, L)])

        fire_gather(0, 0)
        pos_copy.wait()

        @pl.loop(0, ROWS_PER_W + 2, step=2)
        def _(r0):
            for kb in range(2):
                r = r0 + kb
                kp = 1 - kb
                cond = (r >= 1) & (r < ROWS_PER_W + 1)

                @pl.when(cond)
                def _():
                    drain_s1(kp)
                    fire_s2(r - 1, kp)
                    drain_od(kp)

                @pl.when(r + 1 < ROWS_PER_W)
                def _():
                    fire_gather(r + 1, kp)

                @pl.when(r < ROWS_PER_W)
                def _():
                    drain_gather_a(kb)
                    add_pos(kb, 0, W0)
                    pltpu.async_copy(
                        bufs[kb].at[pl.ds(0, W0)],
                        out_hbm.at[pl.ds(wbase + r * S, W0)], odsem[kb])
                    drain_gather_b(kb)
                    add_pos(kb, W0, S)

                    @pl.when(r >= 2)
                    def _():
                        drain_s2(kb)
                    pltpu.async_copy(
                        bufs[kb].at[pl.ds(W0, W1)], shared.at[sid, kb],
                        s1sem[kb])

        drain_s2(0)
        drain_s2(1)

    return k(token_table, pos_table, x_flat)


def kernel(x, token_table, pos_table):
    x_flat = x.reshape(FLAT).astype(jnp.int32)
    out = _sc_embed(x_flat, token_table, pos_table)
    return out.reshape(B, S, D)


# speedup vs baseline: 1.1177x; 1.0113x over previous
import functools

import jax
import jax.numpy as jnp
from jax import lax
from jax.experimental import pallas as pl
from jax.experimental.pallas import tpu as pltpu
from jax.experimental.pallas import tpu_sc as plsc

D = 128
B = 1024
S = 200
L = 16
NC, NS = 2, 16
NW = NC * NS
ROWS_PER_W = B // NW
G0, G1 = 96, 104
W0, W1 = 96, 104
FLAT = B * S


@jax.jit
def _sc_embed(x_flat, token_table, pos_table):
    mesh = plsc.VectorSubcoreMesh(core_axis_name="c", subcore_axis_name="s")

    @functools.partial(
        pl.kernel,
        mesh=mesh,
        out_type=jax.ShapeDtypeStruct((FLAT, D), jnp.float32),
        scratch_types=[
            pltpu.VMEM((S * ROWS_PER_W,), jnp.int32),
            pltpu.VMEM((S, D), jnp.float32),
            pltpu.VMEM_SHARED((NS, 2, W1, D), jnp.float32),
            pltpu.VMEM((S, D), jnp.float32),
            pltpu.VMEM((S, D), jnp.float32),
            pltpu.SemaphoreType.DMA,
            pltpu.SemaphoreType.DMA,
            pltpu.SemaphoreType.DMA,
            pltpu.SemaphoreType.DMA,
            pltpu.SemaphoreType.DMA,
            pltpu.SemaphoreType.DMA,
            pltpu.SemaphoreType.DMA,
            pltpu.SemaphoreType.DMA,
            pltpu.SemaphoreType.DMA,
            pltpu.SemaphoreType.DMA,
            pltpu.SemaphoreType.DMA,
        ],
    )
    def k(tok_hbm, pos_hbm, idx_hbm, out_hbm, idx_v, pos_v, shared,
          buf0, buf1, gsem0, gsem1, gbsem0, gbsem1, s1sem0, s1sem1,
          odsem0, odsem1, s2sem0, s2sem1, psem):
        bufs = (buf0, buf1)
        gsem = (gsem0, gsem1)
        gbsem = (gbsem0, gbsem1)
        s1sem = (s1sem0, s1sem1)
        odsem = (odsem0, odsem1)
        s2sem = (s2sem0, s2sem1)

        sid = lax.axis_index("s")
        wid = sid * NC + lax.axis_index("c")
        wbase = wid * (S * ROWS_PER_W)
        pltpu.sync_copy(idx_hbm.at[pl.ds(wbase, S * ROWS_PER_W)], idx_v)
        pos_copy = pltpu.async_copy(pos_hbm, pos_v, psem)

        def fire_gather(r, kb):
            pltpu.async_copy(
                tok_hbm.at[idx_v.at[pl.ds(r * S, G0)]],
                bufs[kb].at[pl.ds(0, G0)], gsem[kb])
            pltpu.async_copy(
                tok_hbm.at[idx_v.at[pl.ds(r * S + G0, G1)]],
                bufs[kb].at[pl.ds(G0, G1)], gbsem[kb])

        def drain_gather_a(kb):
            pltpu.make_async_copy(
                tok_hbm.at[pl.ds(0, G0)], bufs[kb].at[pl.ds(0, G0)],
                gsem[kb]).wait()

        def drain_gather_b(kb):
            pltpu.make_async_copy(
                tok_hbm.at[pl.ds(0, G1)], bufs[kb].at[pl.ds(G0, G1)],
                gbsem[kb]).wait()

        def drain_od(kb):
            pltpu.make_async_copy(
                bufs[kb].at[pl.ds(0, W0)],
                out_hbm.at[pl.ds(0, W0)], odsem[kb]).wait()

        def drain_s1(kb):
            pltpu.make_async_copy(
                bufs[kb].at[pl.ds(W0, W1)], shared.at[sid, kb],
                s1sem[kb]).wait()

        def fire_s2(r, kb):
            pltpu.async_copy(
                shared.at[sid, kb],
                out_hbm.at[pl.ds(wbase + r * S + W0, W1)], s2sem[kb])

        def drain_s2(kb):
            pltpu.make_async_copy(
                shared.at[sid, kb], out_hbm.at[pl.ds(0, W1)],
                s2sem[kb]).wait()

        def add_pos(kb, lo, hi):
            buf = bufs[kb]

            @pl.loop(lo, hi)
            def _(i):
                for c in range(0, D, L):
                    plsc.addupdate(buf.at[i, pl.ds(c, L)],
                                   pos_v[i, pl.ds(c, L)])

        fire_gather(0, 0)
        pos_copy.wait()

        @pl.loop(0, ROWS_PER_W + 2, step=2)
        def _(r0):
            for kb in range(2):
                r = r0 + kb
                kp = 1 - kb
                cond = (r >= 1) & (r < ROWS_PER_W + 1)

                @pl.when(cond)
                def _():
                    drain_s1(kp)
                    fire_s2(r - 1, kp)
                    drain_od(kp)

                @pl.when(r + 1 < ROWS_PER_W)
                def _():
                    fire_gather(r + 1, kp)

                @pl.when(r < ROWS_PER_W)
                def _():
                    drain_gather_a(kb)
                    add_pos(kb, 0, W0)
                    pltpu.async_copy(
                        bufs[kb].at[pl.ds(0, W0)],
                        out_hbm.at[pl.ds(wbase + r * S, W0)], odsem[kb])
                    drain_gather_b(kb)
                    add_pos(kb, W0, S)

                    @pl.when(r >= 2)
                    def _():
                        drain_s2(kb)
                    pltpu.async_copy(
                        bufs[kb].at[pl.ds(W0, W1)], shared.at[sid, kb],
                        s1sem[kb])

        drain_s2(0)
        drain_s2(1)

    return k(token_table, pos_table, x_flat)


def kernel(x, token_table, pos_table):
    x_flat = x.reshape(FLAT).astype(jnp.int32)
    out = _sc_embed(x_flat, token_table, pos_table)
    return out.reshape(B, S, D)


# speedup vs baseline: 1.1312x; 1.0121x over previous
import functools

import jax
import jax.numpy as jnp
from jax import lax
from jax.experimental import pallas as pl
from jax.experimental.pallas import tpu as pltpu
from jax.experimental.pallas import tpu_sc as plsc

D = 128
B = 1024
S = 200
L = 16
NC, NS = 2, 16
NW = NC * NS
ROWS_PER_W = B // NW
G0, G1 = 104, 96
W0, W1 = 104, 96
FLAT = B * S


@jax.jit
def _sc_embed(x_flat, token_table, pos_table):
    mesh = plsc.VectorSubcoreMesh(core_axis_name="c", subcore_axis_name="s")

    @functools.partial(
        pl.kernel,
        mesh=mesh,
        out_type=jax.ShapeDtypeStruct((FLAT, D), jnp.float32),
        scratch_types=[
            pltpu.VMEM((S * ROWS_PER_W,), jnp.int32),
            pltpu.VMEM((S, D), jnp.float32),
            pltpu.VMEM_SHARED((NS, 2, W1, D), jnp.float32),
            pltpu.VMEM((S, D), jnp.float32),
            pltpu.VMEM((S, D), jnp.float32),
            pltpu.SemaphoreType.DMA,
            pltpu.SemaphoreType.DMA,
            pltpu.SemaphoreType.DMA,
            pltpu.SemaphoreType.DMA,
            pltpu.SemaphoreType.DMA,
            pltpu.SemaphoreType.DMA,
            pltpu.SemaphoreType.DMA,
            pltpu.SemaphoreType.DMA,
            pltpu.SemaphoreType.DMA,
            pltpu.SemaphoreType.DMA,
            pltpu.SemaphoreType.DMA,
        ],
    )
    def k(tok_hbm, pos_hbm, idx_hbm, out_hbm, idx_v, pos_v, shared,
          buf0, buf1, gsem0, gsem1, gbsem0, gbsem1, s1sem0, s1sem1,
          odsem0, odsem1, s2sem0, s2sem1, psem):
        bufs = (buf0, buf1)
        gsem = (gsem0, gsem1)
        gbsem = (gbsem0, gbsem1)
        s1sem = (s1sem0, s1sem1)
        odsem = (odsem0, odsem1)
        s2sem = (s2sem0, s2sem1)

        sid = lax.axis_index("s")
        wid = sid * NC + lax.axis_index("c")
        wbase = wid * (S * ROWS_PER_W)
        pltpu.sync_copy(idx_hbm.at[pl.ds(wbase, S * ROWS_PER_W)], idx_v)
        pos_copy = pltpu.async_copy(pos_hbm, pos_v, psem)

        def fire_gather(r, kb):
            pltpu.async_copy(
                tok_hbm.at[idx_v.at[pl.ds(r * S, G0)]],
                bufs[kb].at[pl.ds(0, G0)], gsem[kb])
            pltpu.async_copy(
                tok_hbm.at[idx_v.at[pl.ds(r * S + G0, G1)]],
                bufs[kb].at[pl.ds(G0, G1)], gbsem[kb])

        def drain_gather_a(kb):
            pltpu.make_async_copy(
                tok_hbm.at[pl.ds(0, G0)], bufs[kb].at[pl.ds(0, G0)],
                gsem[kb]).wait()

        def drain_gather_b(kb):
            pltpu.make_async_copy(
                tok_hbm.at[pl.ds(0, G1)], bufs[kb].at[pl.ds(G0, G1)],
                gbsem[kb]).wait()

        def drain_od(kb):
            pltpu.make_async_copy(
                bufs[kb].at[pl.ds(0, W0)],
                out_hbm.at[pl.ds(0, W0)], odsem[kb]).wait()

        def drain_s1(kb):
            pltpu.make_async_copy(
                bufs[kb].at[pl.ds(W0, W1)], shared.at[sid, kb],
                s1sem[kb]).wait()

        def fire_s2(r, kb):
            pltpu.async_copy(
                shared.at[sid, kb],
                out_hbm.at[pl.ds(wbase + r * S + W0, W1)], s2sem[kb])

        def drain_s2(kb):
            pltpu.make_async_copy(
                shared.at[sid, kb], out_hbm.at[pl.ds(0, W1)],
                s2sem[kb]).wait()

        def add_pos(kb, lo, hi):
            buf = bufs[kb]

            @pl.loop(lo, hi)
            def _(i):
                for c in range(0, D, L):
                    plsc.addupdate(buf.at[i, pl.ds(c, L)],
                                   pos_v[i, pl.ds(c, L)])

        fire_gather(0, 0)
        pos_copy.wait()

        @pl.loop(0, ROWS_PER_W + 2, step=2)
        def _(r0):
            for kb in range(2):
                r = r0 + kb
                kp = 1 - kb
                cond = (r >= 1) & (r < ROWS_PER_W + 1)

                @pl.when(cond)
                def _():
                    drain_s1(kp)
                    fire_s2(r - 1, kp)
                    drain_od(kp)

                @pl.when(r + 1 < ROWS_PER_W)
                def _():
                    fire_gather(r + 1, kp)

                @pl.when(r < ROWS_PER_W)
                def _():
                    drain_gather_a(kb)
                    add_pos(kb, 0, W0)
                    pltpu.async_copy(
                        bufs[kb].at[pl.ds(0, W0)],
                        out_hbm.at[pl.ds(wbase + r * S, W0)], odsem[kb])
                    drain_gather_b(kb)
                    add_pos(kb, W0, S)

                    @pl.when(r >= 2)
                    def _():
                        drain_s2(kb)
                    pltpu.async_copy(
                        bufs[kb].at[pl.ds(W0, W1)], shared.at[sid, kb],
                        s1sem[kb])

        drain_s2(0)
        drain_s2(1)

    return k(token_table, pos_table, x_flat)


def kernel(x, token_table, pos_table):
    x_flat = x.reshape(FLAT).astype(jnp.int32)
    out = _sc_embed(x_flat, token_table, pos_table)
    return out.reshape(B, S, D)


# speedup vs baseline: 1.1414x; 1.0090x over previous
import functools

import jax
import jax.numpy as jnp
from jax import lax
from jax.experimental import pallas as pl
from jax.experimental.pallas import tpu as pltpu
from jax.experimental.pallas import tpu_sc as plsc

D = 128
B = 1024
S = 200
L = 16
NC, NS = 2, 16
NW = NC * NS
ROWS_PER_W = B // NW
G0, G1 = 112, 88
W0, W1 = 112, 88
FLAT = B * S


@jax.jit
def _sc_embed(x_flat, token_table, pos_table):
    mesh = plsc.VectorSubcoreMesh(core_axis_name="c", subcore_axis_name="s")

    @functools.partial(
        pl.kernel,
        mesh=mesh,
        out_type=jax.ShapeDtypeStruct((FLAT, D), jnp.float32),
        scratch_types=[
            pltpu.VMEM((S * ROWS_PER_W,), jnp.int32),
            pltpu.VMEM((S, D), jnp.float32),
            pltpu.VMEM_SHARED((NS, 2, W1, D), jnp.float32),
            pltpu.VMEM((S, D), jnp.float32),
            pltpu.VMEM((S, D), jnp.float32),
            pltpu.SemaphoreType.DMA,
            pltpu.SemaphoreType.DMA,
            pltpu.SemaphoreType.DMA,
            pltpu.SemaphoreType.DMA,
            pltpu.SemaphoreType.DMA,
            pltpu.SemaphoreType.DMA,
            pltpu.SemaphoreType.DMA,
            pltpu.SemaphoreType.DMA,
            pltpu.SemaphoreType.DMA,
            pltpu.SemaphoreType.DMA,
            pltpu.SemaphoreType.DMA,
        ],
    )
    def k(tok_hbm, pos_hbm, idx_hbm, out_hbm, idx_v, pos_v, shared,
          buf0, buf1, gsem0, gsem1, gbsem0, gbsem1, s1sem0, s1sem1,
          odsem0, odsem1, s2sem0, s2sem1, psem):
        bufs = (buf0, buf1)
        gsem = (gsem0, gsem1)
        gbsem = (gbsem0, gbsem1)
        s1sem = (s1sem0, s1sem1)
        odsem = (odsem0, odsem1)
        s2sem = (s2sem0, s2sem1)

        sid = lax.axis_index("s")
        wid = sid * NC + lax.axis_index("c")
        wbase = wid * (S * ROWS_PER_W)
        pltpu.sync_copy(idx_hbm.at[pl.ds(wbase, S * ROWS_PER_W)], idx_v)
        pos_copy = pltpu.async_copy(pos_hbm, pos_v, psem)

        def fire_gather(r, kb):
            pltpu.async_copy(
                tok_hbm.at[idx_v.at[pl.ds(r * S, G0)]],
                bufs[kb].at[pl.ds(0, G0)], gsem[kb])
            pltpu.async_copy(
                tok_hbm.at[idx_v.at[pl.ds(r * S + G0, G1)]],
                bufs[kb].at[pl.ds(G0, G1)], gbsem[kb])

        def drain_gather_a(kb):
            pltpu.make_async_copy(
                tok_hbm.at[pl.ds(0, G0)], bufs[kb].at[pl.ds(0, G0)],
                gsem[kb]).wait()

        def drain_gather_b(kb):
            pltpu.make_async_copy(
                tok_hbm.at[pl.ds(0, G1)], bufs[kb].at[pl.ds(G0, G1)],
                gbsem[kb]).wait()

        def drain_od(kb):
            pltpu.make_async_copy(
                bufs[kb].at[pl.ds(0, W0)],
                out_hbm.at[pl.ds(0, W0)], odsem[kb]).wait()

        def drain_s1(kb):
            pltpu.make_async_copy(
                bufs[kb].at[pl.ds(W0, W1)], shared.at[sid, kb],
                s1sem[kb]).wait()

        def fire_s2(r, kb):
            pltpu.async_copy(
                shared.at[sid, kb],
                out_hbm.at[pl.ds(wbase + r * S + W0, W1)], s2sem[kb])

        def drain_s2(kb):
            pltpu.make_async_copy(
                shared.at[sid, kb], out_hbm.at[pl.ds(0, W1)],
                s2sem[kb]).wait()

        def add_pos(kb, lo, hi):
            buf = bufs[kb]

            @pl.loop(lo, hi)
            def _(i):
                for c in range(0, D, L):
                    plsc.addupdate(buf.at[i, pl.ds(c, L)],
                                   pos_v[i, pl.ds(c, L)])

        fire_gather(0, 0)
        pos_copy.wait()

        @pl.loop(0, ROWS_PER_W + 2, step=2)
        def _(r0):
            for kb in range(2):
                r = r0 + kb
                kp = 1 - kb
                cond = (r >= 1) & (r < ROWS_PER_W + 1)

                @pl.when(cond)
                def _():
                    drain_s1(kp)
                    fire_s2(r - 1, kp)
                    drain_od(kp)

                @pl.when(r + 1 < ROWS_PER_W)
                def _():
                    fire_gather(r + 1, kp)

                @pl.when(r < ROWS_PER_W)
                def _():
                    drain_gather_a(kb)
                    add_pos(kb, 0, W0)
                    pltpu.async_copy(
                        bufs[kb].at[pl.ds(0, W0)],
                        out_hbm.at[pl.ds(wbase + r * S, W0)], odsem[kb])
                    drain_gather_b(kb)
                    add_pos(kb, W0, S)

                    @pl.when(r >= 2)
                    def _():
                        drain_s2(kb)
                    pltpu.async_copy(
                        bufs[kb].at[pl.ds(W0, W1)], shared.at[sid, kb],
                        s1sem[kb])

        drain_s2(0)
        drain_s2(1)

    return k(token_table, pos_table, x_flat)


def kernel(x, token_table, pos_table):
    x_flat = x.reshape(FLAT).astype(jnp.int32)
    out = _sc_embed(x_flat, token_table, pos_table)
    return out.reshape(B, S, D)


# speedup vs baseline: 1.1495x; 1.0071x over previous
import functools

import jax
import jax.numpy as jnp
from jax import lax
from jax.experimental import pallas as pl
from jax.experimental.pallas import tpu as pltpu
from jax.experimental.pallas import tpu_sc as plsc

D = 128
B = 1024
S = 200
L = 16
NC, NS = 2, 16
NW = NC * NS
ROWS_PER_W = B // NW
G0, G1 = 120, 80
W0, W1 = 120, 80
FLAT = B * S


@jax.jit
def _sc_embed(x_flat, token_table, pos_table):
    mesh = plsc.VectorSubcoreMesh(core_axis_name="c", subcore_axis_name="s")

    @functools.partial(
        pl.kernel,
        mesh=mesh,
        out_type=jax.ShapeDtypeStruct((FLAT, D), jnp.float32),
        scratch_types=[
            pltpu.VMEM((S * ROWS_PER_W,), jnp.int32),
            pltpu.VMEM((S, D), jnp.float32),
            pltpu.VMEM_SHARED((NS, 2, W1, D), jnp.float32),
            pltpu.VMEM((S, D), jnp.float32),
            pltpu.VMEM((S, D), jnp.float32),
            pltpu.SemaphoreType.DMA,
            pltpu.SemaphoreType.DMA,
            pltpu.SemaphoreType.DMA,
            pltpu.SemaphoreType.DMA,
            pltpu.SemaphoreType.DMA,
            pltpu.SemaphoreType.DMA,
            pltpu.SemaphoreType.DMA,
            pltpu.SemaphoreType.DMA,
            pltpu.SemaphoreType.DMA,
            pltpu.SemaphoreType.DMA,
            pltpu.SemaphoreType.DMA,
        ],
    )
    def k(tok_hbm, pos_hbm, idx_hbm, out_hbm, idx_v, pos_v, shared,
          buf0, buf1, gsem0, gsem1, gbsem0, gbsem1, s1sem0, s1sem1,
          odsem0, odsem1, s2sem0, s2sem1, psem):
        bufs = (buf0, buf1)
        gsem = (gsem0, gsem1)
        gbsem = (gbsem0, gbsem1)
        s1sem = (s1sem0, s1sem1)
        odsem = (odsem0, odsem1)
        s2sem = (s2sem0, s2sem1)

        sid = lax.axis_index("s")
        wid = sid * NC + lax.axis_index("c")
        wbase = wid * (S * ROWS_PER_W)
        pltpu.sync_copy(idx_hbm.at[pl.ds(wbase, S * ROWS_PER_W)], idx_v)
        pos_copy = pltpu.async_copy(pos_hbm, pos_v, psem)

        def fire_gather(r, kb):
            pltpu.async_copy(
                tok_hbm.at[idx_v.at[pl.ds(r * S, G0)]],
                bufs[kb].at[pl.ds(0, G0)], gsem[kb])
            pltpu.async_copy(
                tok_hbm.at[idx_v.at[pl.ds(r * S + G0, G1)]],
                bufs[kb].at[pl.ds(G0, G1)], gbsem[kb])

        def drain_gather_a(kb):
            pltpu.make_async_copy(
                tok_hbm.at[pl.ds(0, G0)], bufs[kb].at[pl.ds(0, G0)],
                gsem[kb]).wait()

        def drain_gather_b(kb):
            pltpu.make_async_copy(
                tok_hbm.at[pl.ds(0, G1)], bufs[kb].at[pl.ds(G0, G1)],
                gbsem[kb]).wait()

        def drain_od(kb):
            pltpu.make_async_copy(
                bufs[kb].at[pl.ds(0, W0)],
                out_hbm.at[pl.ds(0, W0)], odsem[kb]).wait()

        def drain_s1(kb):
            pltpu.make_async_copy(
                bufs[kb].at[pl.ds(W0, W1)], shared.at[sid, kb],
                s1sem[kb]).wait()

        def fire_s2(r, kb):
            pltpu.async_copy(
                shared.at[sid, kb],
                out_hbm.at[pl.ds(wbase + r * S + W0, W1)], s2sem[kb])

        def drain_s2(kb):
            pltpu.make_async_copy(
                shared.at[sid, kb], out_hbm.at[pl.ds(0, W1)],
                s2sem[kb]).wait()

        def add_pos(kb, lo, hi):
            buf = bufs[kb]

            @pl.loop(lo, hi)
            def _(i):
                for c in range(0, D, L):
                    plsc.addupdate(buf.at[i, pl.ds(c, L)],
                                   pos_v[i, pl.ds(c, L)])

        fire_gather(0, 0)
        pos_copy.wait()

        @pl.loop(0, ROWS_PER_W + 2, step=2)
        def _(r0):
            for kb in range(2):
                r = r0 + kb
                kp = 1 - kb
                cond = (r >= 1) & (r < ROWS_PER_W + 1)

                @pl.when(cond)
                def _():
                    drain_s1(kp)
                    fire_s2(r - 1, kp)
                    drain_od(kp)

                @pl.when(r + 1 < ROWS_PER_W)
                def _():
                    fire_gather(r + 1, kp)

                @pl.when(r < ROWS_PER_W)
                def _():
                    drain_gather_a(kb)
                    add_pos(kb, 0, W0)
                    pltpu.async_copy(
                        bufs[kb].at[pl.ds(0, W0)],
                        out_hbm.at[pl.ds(wbase + r * S, W0)], odsem[kb])
                    drain_gather_b(kb)
                    add_pos(kb, W0, S)

                    @pl.when(r >= 2)
                    def _():
                        drain_s2(kb)
                    pltpu.async_copy(
                        bufs[kb].at[pl.ds(W0, W1)], shared.at[sid, kb],
                        s1sem[kb])

        drain_s2(0)
        drain_s2(1)

    return k(token_table, pos_table, x_flat)


def kernel(x, token_table, pos_table):
    x_flat = x.reshape(FLAT).astype(jnp.int32)
    out = _sc_embed(x_flat, token_table, pos_table)
    return out.reshape(B, S, D)


# speedup vs baseline: 1.1513x; 1.0016x over previous
import functools

import jax
import jax.numpy as jnp
from jax import lax
from jax.experimental import pallas as pl
from jax.experimental.pallas import tpu as pltpu
from jax.experimental.pallas import tpu_sc as plsc

D = 128
B = 1024
S = 200
L = 16
NC, NS = 2, 16
NW = NC * NS
ROWS_PER_W = B // NW
G0, G1 = 128, 72
W0, W1 = 128, 72
FLAT = B * S


@jax.jit
def _sc_embed(x_flat, token_table, pos_table):
    mesh = plsc.VectorSubcoreMesh(core_axis_name="c", subcore_axis_name="s")

    @functools.partial(
        pl.kernel,
        mesh=mesh,
        out_type=jax.ShapeDtypeStruct((FLAT, D), jnp.float32),
        scratch_types=[
            pltpu.VMEM((S * ROWS_PER_W,), jnp.int32),
            pltpu.VMEM((S, D), jnp.float32),
            pltpu.VMEM_SHARED((NS, 2, W1, D), jnp.float32),
            pltpu.VMEM((S, D), jnp.float32),
            pltpu.VMEM((S, D), jnp.float32),
            pltpu.SemaphoreType.DMA,
            pltpu.SemaphoreType.DMA,
            pltpu.SemaphoreType.DMA,
            pltpu.SemaphoreType.DMA,
            pltpu.SemaphoreType.DMA,
            pltpu.SemaphoreType.DMA,
            pltpu.SemaphoreType.DMA,
            pltpu.SemaphoreType.DMA,
            pltpu.SemaphoreType.DMA,
            pltpu.SemaphoreType.DMA,
            pltpu.SemaphoreType.DMA,
        ],
    )
    def k(tok_hbm, pos_hbm, idx_hbm, out_hbm, idx_v, pos_v, shared,
          buf0, buf1, gsem0, gsem1, gbsem0, gbsem1, s1sem0, s1sem1,
          odsem0, odsem1, s2sem0, s2sem1, psem):
        bufs = (buf0, buf1)
        gsem = (gsem0, gsem1)
        gbsem = (gbsem0, gbsem1)
        s1sem = (s1sem0, s1sem1)
        odsem = (odsem0, odsem1)
        s2sem = (s2sem0, s2sem1)

        sid = lax.axis_index("s")
        wid = sid * NC + lax.axis_index("c")
        wbase = wid * (S * ROWS_PER_W)
        pltpu.sync_copy(idx_hbm.at[pl.ds(wbase, S * ROWS_PER_W)], idx_v)
        pos_copy = pltpu.async_copy(pos_hbm, pos_v, psem)

        def fire_gather(r, kb):
            pltpu.async_copy(
                tok_hbm.at[idx_v.at[pl.ds(r * S, G0)]],
                bufs[kb].at[pl.ds(0, G0)], gsem[kb])
            pltpu.async_copy(
                tok_hbm.at[idx_v.at[pl.ds(r * S + G0, G1)]],
                bufs[kb].at[pl.ds(G0, G1)], gbsem[kb])

        def drain_gather_a(kb):
            pltpu.make_async_copy(
                tok_hbm.at[pl.ds(0, G0)], bufs[kb].at[pl.ds(0, G0)],
                gsem[kb]).wait()

        def drain_gather_b(kb):
            pltpu.make_async_copy(
                tok_hbm.at[pl.ds(0, G1)], bufs[kb].at[pl.ds(G0, G1)],
                gbsem[kb]).wait()

        def drain_od(kb):
            pltpu.make_async_copy(
                bufs[kb].at[pl.ds(0, W0)],
                out_hbm.at[pl.ds(0, W0)], odsem[kb]).wait()

        def drain_s1(kb):
            pltpu.make_async_copy(
                bufs[kb].at[pl.ds(W0, W1)], shared.at[sid, kb],
                s1sem[kb]).wait()

        def fire_s2(r, kb):
            pltpu.async_copy(
                shared.at[sid, kb],
                out_hbm.at[pl.ds(wbase + r * S + W0, W1)], s2sem[kb])

        def drain_s2(kb):
            pltpu.make_async_copy(
                shared.at[sid, kb], out_hbm.at[pl.ds(0, W1)],
                s2sem[kb]).wait()

        def add_pos(kb, lo, hi):
            buf = bufs[kb]

            @pl.loop(lo, hi)
            def _(i):
                for c in range(0, D, L):
                    plsc.addupdate(buf.at[i, pl.ds(c, L)],
                                   pos_v[i, pl.ds(c, L)])

        fire_gather(0, 0)
        pos_copy.wait()

        @pl.loop(0, ROWS_PER_W + 2, step=2)
        def _(r0):
            for kb in range(2):
                r = r0 + kb
                kp = 1 - kb
                cond = (r >= 1) & (r < ROWS_PER_W + 1)

                @pl.when(cond)
                def _():
                    drain_s1(kp)
                    fire_s2(r - 1, kp)
                    drain_od(kp)

                @pl.when(r + 1 < ROWS_PER_W)
                def _():
                    fire_gather(r + 1, kp)

                @pl.when(r < ROWS_PER_W)
                def _():
                    drain_gather_a(kb)
                    add_pos(kb, 0, W0)
                    pltpu.async_copy(
                        bufs[kb].at[pl.ds(0, W0)],
                        out_hbm.at[pl.ds(wbase + r * S, W0)], odsem[kb])
                    drain_gather_b(kb)
                    add_pos(kb, W0, S)

                    @pl.when(r >= 2)
                    def _():
                        drain_s2(kb)
                    pltpu.async_copy(
                        bufs[kb].at[pl.ds(W0, W1)], shared.at[sid, kb],
                        s1sem[kb])

        drain_s2(0)
        drain_s2(1)

    return k(token_table, pos_table, x_flat)


def kernel(x, token_table, pos_table):
    x_flat = x.reshape(FLAT).astype(jnp.int32)
    out = _sc_embed(x_flat, token_table, pos_table)
    return out.reshape(B, S, D)


# speedup vs baseline: 1.1527x; 1.0012x over previous
import functools

import jax
import jax.numpy as jnp
from jax import lax
from jax.experimental import pallas as pl
from jax.experimental.pallas import tpu as pltpu
from jax.experimental.pallas import tpu_sc as plsc

D = 128
B = 1024
S = 200
L = 16
NC, NS = 2, 16
NW = NC * NS
ROWS_PER_W = B // NW
G0, G1 = 128, 72
W0, W1 = 136, 64
FLAT = B * S


@jax.jit
def _sc_embed(x_flat, token_table, pos_table):
    mesh = plsc.VectorSubcoreMesh(core_axis_name="c", subcore_axis_name="s")

    @functools.partial(
        pl.kernel,
        mesh=mesh,
        out_type=jax.ShapeDtypeStruct((FLAT, D), jnp.float32),
        scratch_types=[
            pltpu.VMEM((S * ROWS_PER_W,), jnp.int32),
            pltpu.VMEM((S, D), jnp.float32),
            pltpu.VMEM_SHARED((NS, 2, W1, D), jnp.float32),
            pltpu.VMEM((S, D), jnp.float32),
            pltpu.VMEM((S, D), jnp.float32),
            pltpu.SemaphoreType.DMA,
            pltpu.SemaphoreType.DMA,
            pltpu.SemaphoreType.DMA,
            pltpu.SemaphoreType.DMA,
            pltpu.SemaphoreType.DMA,
            pltpu.SemaphoreType.DMA,
            pltpu.SemaphoreType.DMA,
            pltpu.SemaphoreType.DMA,
            pltpu.SemaphoreType.DMA,
            pltpu.SemaphoreType.DMA,
            pltpu.SemaphoreType.DMA,
        ],
    )
    def k(tok_hbm, pos_hbm, idx_hbm, out_hbm, idx_v, pos_v, shared,
          buf0, buf1, gsem0, gsem1, gbsem0, gbsem1, s1sem0, s1sem1,
          odsem0, odsem1, s2sem0, s2sem1, psem):
        bufs = (buf0, buf1)
        gsem = (gsem0, gsem1)
        gbsem = (gbsem0, gbsem1)
        s1sem = (s1sem0, s1sem1)
        odsem = (odsem0, odsem1)
        s2sem = (s2sem0, s2sem1)

        sid = lax.axis_index("s")
        wid = sid * NC + lax.axis_index("c")
        wbase = wid * (S * ROWS_PER_W)
        pltpu.sync_copy(idx_hbm.at[pl.ds(wbase, S * ROWS_PER_W)], idx_v)
        pos_copy = pltpu.async_copy(pos_hbm, pos_v, psem)

        def fire_gather(r, kb):
            pltpu.async_copy(
                tok_hbm.at[idx_v.at[pl.ds(r * S, G0)]],
                bufs[kb].at[pl.ds(0, G0)], gsem[kb])
            pltpu.async_copy(
                tok_hbm.at[idx_v.at[pl.ds(r * S + G0, G1)]],
                bufs[kb].at[pl.ds(G0, G1)], gbsem[kb])

        def drain_gather_a(kb):
            pltpu.make_async_copy(
                tok_hbm.at[pl.ds(0, G0)], bufs[kb].at[pl.ds(0, G0)],
                gsem[kb]).wait()

        def drain_gather_b(kb):
            pltpu.make_async_copy(
                tok_hbm.at[pl.ds(0, G1)], bufs[kb].at[pl.ds(G0, G1)],
                gbsem[kb]).wait()

        def drain_od(kb):
            pltpu.make_async_copy(
                bufs[kb].at[pl.ds(0, W0)],
                out_hbm.at[pl.ds(0, W0)], odsem[kb]).wait()

        def drain_s1(kb):
            pltpu.make_async_copy(
                bufs[kb].at[pl.ds(W0, W1)], shared.at[sid, kb],
                s1sem[kb]).wait()

        def fire_s2(r, kb):
            pltpu.async_copy(
                shared.at[sid, kb],
                out_hbm.at[pl.ds(wbase + r * S + W0, W1)], s2sem[kb])

        def drain_s2(kb):
            pltpu.make_async_copy(
                shared.at[sid, kb], out_hbm.at[pl.ds(0, W1)],
                s2sem[kb]).wait()

        def add_pos(kb, lo, hi):
            buf = bufs[kb]

            @pl.loop(lo, hi)
            def _(i):
                for c in range(0, D, L):
                    plsc.addupdate(buf.at[i, pl.ds(c, L)],
                                   pos_v[i, pl.ds(c, L)])

        fire_gather(0, 0)
        pos_copy.wait()

        @pl.loop(0, ROWS_PER_W + 2, step=2)
        def _(r0):
            for kb in range(2):
                r = r0 + kb
                kp = 1 - kb
                cond = (r >= 1) & (r < ROWS_PER_W + 1)

                @pl.when(cond)
                def _():
                    drain_s1(kp)
                    fire_s2(r - 1, kp)
                    drain_od(kp)

                @pl.when(r + 1 < ROWS_PER_W)
                def _():
                    fire_gather(r + 1, kp)

                @pl.when(r < ROWS_PER_W)
                def _():
                    drain_gather_a(kb)
                    add_pos(kb, 0, min(G0, W0))
                    if W0 > G0:
                        drain_gather_b(kb)
                        add_pos(kb, G0, W0)
                    pltpu.async_copy(
                        bufs[kb].at[pl.ds(0, W0)],
                        out_hbm.at[pl.ds(wbase + r * S, W0)], odsem[kb])
                    if W0 <= G0:
                        drain_gather_b(kb)
                    add_pos(kb, W0, S)

                    @pl.when(r >= 2)
                    def _():
                        drain_s2(kb)
                    pltpu.async_copy(
                        bufs[kb].at[pl.ds(W0, W1)], shared.at[sid, kb],
                        s1sem[kb])

        drain_s2(0)
        drain_s2(1)

    return k(token_table, pos_table, x_flat)


def kernel(x, token_table, pos_table):
    x_flat = x.reshape(FLAT).astype(jnp.int32)
    out = _sc_embed(x_flat, token_table, pos_table)
    return out.reshape(B, S, D)
